# baseline (speedup 1.0000x reference)
import jax
import jax.numpy as jnp
from jax import lax
from jax.experimental import pallas as pl
from jax.experimental.pallas import tpu as pltpu
from jax.experimental.pallas import tpu_sc as plsc

B, L, D = 16384, 50, 32
NC, NS = 2, 16
NW = NC * NS
BAGS_PER_W = B // NW
T = 32
N_IT = BAGS_PER_W // T
IDX_CHUNK = T * L
INV_L = 1.0 / L


def _body(idx_hbm, w_hbm, out_hbm, idx_v0, idx_v1, rows_v0, rows_v1, out_v,
          sem0, sem1):
    wid = lax.axis_index("s") * NC + lax.axis_index("c")
    base_bag = wid * BAGS_PER_W
    idx_bufs = (idx_v0, idx_v1)
    rows_bufs = (rows_v0, rows_v1)
    sems = (sem0, sem1)

    NSPLIT = 8
    SPLIT = IDX_CHUNK // NSPLIT

    def start_gather(t, slot):
        bag0 = base_bag + t * T
        pltpu.sync_copy(idx_hbm.at[pl.ds(bag0 * L, IDX_CHUNK)],
                        idx_bufs[slot])
        return [
            pltpu.async_copy(
                w_hbm.at[idx_bufs[slot].at[pl.ds(s * SPLIT, SPLIT)]],
                rows_bufs[slot].at[pl.ds(s * SPLIT, SPLIT)],
                sems[slot])
            for s in range(NSPLIT)]

    copies = [None, None]
    copies[0] = start_gather(0, 0)
    for t in range(N_IT):
        cur = t % 2
        if t + 1 < N_IT:
            copies[(t + 1) % 2] = start_gather(t + 1, (t + 1) % 2)
        for c in copies[cur]:
            c.wait()
        rows_v = rows_bufs[cur]

        @pl.loop(0, T, unroll=2)
        def _bag(b):
            r0 = b * L
            a0 = rows_v[r0, pl.ds(0, 16)]
            b0 = rows_v[r0 + 1, pl.ds(0, 16)]
            a1 = rows_v[r0, pl.ds(16, 16)]
            b1 = rows_v[r0 + 1, pl.ds(16, 16)]
            for j in range(2, L, 2):
                a0 = a0 + rows_v[r0 + j, pl.ds(0, 16)]
                b0 = b0 + rows_v[r0 + j + 1, pl.ds(0, 16)]
                a1 = a1 + rows_v[r0 + j, pl.ds(16, 16)]
                b1 = b1 + rows_v[r0 + j + 1, pl.ds(16, 16)]
            out_v[b, pl.ds(0, 16)] = (a0 + b0) * INV_L
            out_v[b, pl.ds(16, 16)] = (a1 + b1) * INV_L

        pltpu.sync_copy(out_v, out_hbm.at[pl.ds(base_bag + t * T, T), :])


@jax.jit
def kernel(inputs, weights):
    flat_idx = inputs.reshape(-1)
    mesh = plsc.VectorSubcoreMesh(
        core_axis_name="c", subcore_axis_name="s",
        num_cores=NC, num_subcores=NS)
    k = pl.kernel(
        _body,
        out_type=jax.ShapeDtypeStruct((B, D), jnp.float32),
        mesh=mesh,
        scratch_types=[
            pltpu.VMEM((IDX_CHUNK,), jnp.int32),
            pltpu.VMEM((IDX_CHUNK,), jnp.int32),
            pltpu.VMEM((IDX_CHUNK, D), jnp.float32),
            pltpu.VMEM((IDX_CHUNK, D), jnp.float32),
            pltpu.VMEM((T, D), jnp.float32),
            pltpu.SemaphoreType.DMA,
            pltpu.SemaphoreType.DMA,
        ],
        compiler_params=pltpu.CompilerParams(use_tc_tiling_on_sc=False),
    )
    return k(flat_idx, weights)
